# baseline (speedup 1.0000x reference)
import jax
import jax.numpy as jnp
from jax import lax
from jax.experimental import pallas as pl
from jax.experimental.pallas import tpu as pltpu
from jax.experimental.pallas import tpu_sc as plsc

B, N, D = 4096, 50, 256
NC, NS = 2, 16
NW = NC * NS
BPW = (B // 2) // NW
LANES = 16


def _sc_gather_body(char_id, style_id, char_t, style_t, char_m, style_m,
                    char_out, style_out, idx, bufa, bufb, sem_a, sem_b):
    wid = lax.axis_index("s") * NC + lax.axis_index("c")
    base = wid * BPW
    for id_ref, t_ref, m_ref, out_ref in (
        (char_id, char_t, char_m, char_out),
        (style_id, style_t, style_m, style_out),
    ):
        pltpu.sync_copy(id_ref.at[pl.ds(base, BPW)], idx)
        cp_a = pltpu.async_copy(t_ref.at[idx], bufa, sem_a)
        cp_b = pltpu.async_copy(m_ref.at[idx], bufb, sem_b)
        cp_a.wait()
        cp_b.wait()

        def add_row(r, _):
            for c in range(D // LANES):
                sl = pl.ds(c * LANES, LANES)
                bufa[r, sl] = bufa[r, sl] + bufb[r, sl]
            return 0

        lax.fori_loop(0, BPW, add_row, 0)
        pltpu.sync_copy(bufa, out_ref.at[pl.ds(base, BPW)])


def _sc_gather(char_id, style_id, char_t, style_t, char_m, style_m):
    mesh = plsc.VectorSubcoreMesh(core_axis_name="c", subcore_axis_name="s",
                                  num_cores=NC, num_subcores=NS)
    return pl.kernel(
        _sc_gather_body,
        out_type=(
            jax.ShapeDtypeStruct((B // 2, D), jnp.float32),
            jax.ShapeDtypeStruct((B // 2, D), jnp.float32),
        ),
        mesh=mesh,
        scratch_types=[
            pltpu.VMEM((BPW,), jnp.int32),
            pltpu.VMEM((BPW, D), jnp.float32),
            pltpu.VMEM((BPW, D), jnp.float32),
            pltpu.SemaphoreType.DMA,
            pltpu.SemaphoreType.DMA,
        ],
    )(char_id, style_id, char_t, style_t, char_m, style_m)


BB = 256


def _tc_body(char_ref, style_ref, w1_ref, b1_ref, w2_ref, b2_ref,
             g_ref, be_ref, x_ref, out_ref):
    cc = char_ref[...]
    cs = style_ref[...]
    h = (
        jnp.dot(cc, w1_ref[0:D, :], preferred_element_type=jnp.float32,
                precision=lax.Precision.HIGHEST)
        + jnp.dot(cs, w1_ref[D:2 * D, :], preferred_element_type=jnp.float32,
                  precision=lax.Precision.HIGHEST)
        + b1_ref[...]
    )
    h = 0.5 * h * (1.0 + lax.erf(h * (2.0 ** -0.5)))
    h = jnp.dot(h, w2_ref[...], preferred_element_type=jnp.float32,
                precision=lax.Precision.HIGHEST) + b2_ref[...]
    mu = jnp.mean(h, axis=-1, keepdims=True)
    hc = h - mu
    var = jnp.mean(hc * hc, axis=-1, keepdims=True)
    hn = hc * lax.rsqrt(var + 1e-5)
    cp = hn * g_ref[...] + be_ref[...]
    out_ref[...] = x_ref[...] + 0.3 * cp[None, :, :]


def _tc_tail(char_comb, style_comb, W1, b1, W2, b2, ln_g, ln_b, x):
    xt = jnp.transpose(x, (1, 0, 2))
    vec = lambda v: v.reshape(1, D)
    out_t = pl.pallas_call(
        _tc_body,
        grid=(B // BB,),
        in_specs=[
            pl.BlockSpec((BB, D), lambda i: (i, 0)),
            pl.BlockSpec((BB, D), lambda i: (i, 0)),
            pl.BlockSpec((2 * D, D), lambda i: (0, 0)),
            pl.BlockSpec((1, D), lambda i: (0, 0)),
            pl.BlockSpec((D, D), lambda i: (0, 0)),
            pl.BlockSpec((1, D), lambda i: (0, 0)),
            pl.BlockSpec((1, D), lambda i: (0, 0)),
            pl.BlockSpec((1, D), lambda i: (0, 0)),
            pl.BlockSpec((N, BB, D), lambda i: (0, i, 0)),
        ],
        out_specs=pl.BlockSpec((N, BB, D), lambda i: (0, i, 0)),
        out_shape=jax.ShapeDtypeStruct((N, B, D), jnp.float32),
    )(char_comb, style_comb, W1, vec(b1), W2, vec(b2), vec(ln_g), vec(ln_b),
      xt)
    return jnp.transpose(out_t, (1, 0, 2))


HB = B // 2
NBLK = HB // BB


def _tc_body_sel(cc0_ref, cs0_ref, cc1_ref, cs1_ref, w1_ref, b1_ref,
                 w2_ref, b2_ref, g_ref, be_ref, x_ref, out_ref):
    in_h1 = pl.program_id(0) == 1
    cc = jnp.where(in_h1, cc1_ref[...], cc0_ref[...])
    cs = jnp.where(in_h1, cs1_ref[...], cs0_ref[...])
    h = (
        jnp.dot(cc, w1_ref[0:D, :], preferred_element_type=jnp.float32,
                precision=lax.Precision.HIGHEST)
        + jnp.dot(cs, w1_ref[D:2 * D, :], preferred_element_type=jnp.float32,
                  precision=lax.Precision.HIGHEST)
        + b1_ref[...]
    )
    h = 0.5 * h * (1.0 + lax.erf(h * (2.0 ** -0.5)))
    h = jnp.dot(h, w2_ref[...], preferred_element_type=jnp.float32,
                precision=lax.Precision.HIGHEST) + b2_ref[...]
    mu = jnp.mean(h, axis=-1, keepdims=True)
    hc = h - mu
    var = jnp.mean(hc * hc, axis=-1, keepdims=True)
    hn = hc * lax.rsqrt(var + 1e-5)
    cp = hn * g_ref[...] + be_ref[...]
    out_ref[...] = x_ref[...] + 0.3 * cp[None, :, :]


def _tc_tail2(cc0, cs0, cc1, cs1, W1, b1, W2, b2, ln_g, ln_b, x):
    xt = jnp.transpose(x, (1, 0, 2))
    vec = lambda v: v.reshape(1, D)
    out_t = pl.pallas_call(
        _tc_body_sel,
        grid=(2, NBLK),
        in_specs=[
            pl.BlockSpec((BB, D), lambda h, i: (i * (1 - h), 0)),
            pl.BlockSpec((BB, D), lambda h, i: (i * (1 - h), 0)),
            pl.BlockSpec((BB, D), lambda h, i: (i * h, 0)),
            pl.BlockSpec((BB, D), lambda h, i: (i * h, 0)),
            pl.BlockSpec((2 * D, D), lambda h, i: (0, 0)),
            pl.BlockSpec((1, D), lambda h, i: (0, 0)),
            pl.BlockSpec((D, D), lambda h, i: (0, 0)),
            pl.BlockSpec((1, D), lambda h, i: (0, 0)),
            pl.BlockSpec((1, D), lambda h, i: (0, 0)),
            pl.BlockSpec((1, D), lambda h, i: (0, 0)),
            pl.BlockSpec((N, BB, D), lambda h, i: (0, h * NBLK + i, 0)),
        ],
        out_specs=pl.BlockSpec((N, BB, D), lambda h, i: (0, h * NBLK + i, 0)),
        out_shape=jax.ShapeDtypeStruct((N, B, D), jnp.float32),
    )(cc0, cs0, cc1, cs1, W1, vec(b1), W2, vec(b2), vec(ln_g), vec(ln_b),
      xt)
    return jnp.transpose(out_t, (1, 0, 2))


def kernel(x, character_id, style_id, char_table, style_table, char_memory,
           style_memory, W1, b1, W2, b2, ln_g, ln_b):
    cc0, cs0 = _sc_gather(character_id[:HB], style_id[:HB], char_table,
                          style_table, char_memory, style_memory)
    cc1, cs1 = _sc_gather(character_id[HB:], style_id[HB:], char_table,
                          style_table, char_memory, style_memory)
    return _tc_tail2(cc0, cs0, cc1, cs1, W1, b1, W2, b2, ln_g, ln_b, x)


# speedup vs baseline: 1.0381x; 1.0381x over previous
import jax
import jax.numpy as jnp
from jax import lax
from jax.experimental import pallas as pl
from jax.experimental.pallas import tpu as pltpu
from jax.experimental.pallas import tpu_sc as plsc

B, N, D = 4096, 50, 256
NC, NS = 2, 16
NW = NC * NS
BPW = B // NW
LANES = 16


def _sc_gather_body(char_id, style_id, char_t, style_t, char_m, style_m,
                    char_out, style_out, idx, bufa, bufb, sem_a, sem_b):
    wid = lax.axis_index("s") * NC + lax.axis_index("c")
    base = wid * BPW
    for id_ref, t_ref, m_ref, out_ref in (
        (char_id, char_t, char_m, char_out),
        (style_id, style_t, style_m, style_out),
    ):
        pltpu.sync_copy(id_ref.at[pl.ds(base, BPW)], idx)
        cp_a = pltpu.async_copy(t_ref.at[idx], bufa, sem_a)
        cp_b = pltpu.async_copy(m_ref.at[idx], bufb, sem_b)
        cp_a.wait()
        cp_b.wait()

        def add_row(r, _):
            for c in range(D // LANES):
                sl = pl.ds(c * LANES, LANES)
                bufa[r, sl] = bufa[r, sl] + bufb[r, sl]
            return 0

        lax.fori_loop(0, BPW, add_row, 0)
        pltpu.sync_copy(bufa, out_ref.at[pl.ds(base, BPW)])


def _sc_gather(char_id, style_id, char_t, style_t, char_m, style_m):
    mesh = plsc.VectorSubcoreMesh(core_axis_name="c", subcore_axis_name="s",
                                  num_cores=NC, num_subcores=NS)
    return pl.kernel(
        _sc_gather_body,
        out_type=(
            jax.ShapeDtypeStruct((B, D), jnp.float32),
            jax.ShapeDtypeStruct((B, D), jnp.float32),
        ),
        mesh=mesh,
        scratch_types=[
            pltpu.VMEM((BPW,), jnp.int32),
            pltpu.VMEM((BPW, D), jnp.float32),
            pltpu.VMEM((BPW, D), jnp.float32),
            pltpu.SemaphoreType.DMA,
            pltpu.SemaphoreType.DMA,
        ],
    )(char_id, style_id, char_t, style_t, char_m, style_m)


BB = 256


def _tc_body(char_ref, style_ref, w1_ref, b1_ref, w2_ref, b2_ref,
             g_ref, be_ref, x_ref, out_ref):
    cc = char_ref[...]
    cs = style_ref[...]
    h = (
        jnp.dot(cc, w1_ref[0:D, :], preferred_element_type=jnp.float32,
                precision=lax.Precision.HIGHEST)
        + jnp.dot(cs, w1_ref[D:2 * D, :], preferred_element_type=jnp.float32,
                  precision=lax.Precision.HIGHEST)
        + b1_ref[...]
    )
    h = 0.5 * h * (1.0 + lax.erf(h * (2.0 ** -0.5)))
    h = jnp.dot(h, w2_ref[...], preferred_element_type=jnp.float32,
                precision=lax.Precision.HIGHEST) + b2_ref[...]
    mu = jnp.mean(h, axis=-1, keepdims=True)
    hc = h - mu
    var = jnp.mean(hc * hc, axis=-1, keepdims=True)
    hn = hc * lax.rsqrt(var + 1e-5)
    cp = hn * g_ref[...] + be_ref[...]
    out_ref[...] = x_ref[...] + 0.3 * cp[None, :, :]


def _tc_tail(char_comb, style_comb, W1, b1, W2, b2, ln_g, ln_b, x):
    xt = jnp.transpose(x, (1, 0, 2))
    vec = lambda v: v.reshape(1, D)
    out_t = pl.pallas_call(
        _tc_body,
        grid=(B // BB,),
        in_specs=[
            pl.BlockSpec((BB, D), lambda i: (i, 0)),
            pl.BlockSpec((BB, D), lambda i: (i, 0)),
            pl.BlockSpec((2 * D, D), lambda i: (0, 0)),
            pl.BlockSpec((1, D), lambda i: (0, 0)),
            pl.BlockSpec((D, D), lambda i: (0, 0)),
            pl.BlockSpec((1, D), lambda i: (0, 0)),
            pl.BlockSpec((1, D), lambda i: (0, 0)),
            pl.BlockSpec((1, D), lambda i: (0, 0)),
            pl.BlockSpec((N, BB, D), lambda i: (0, i, 0)),
        ],
        out_specs=pl.BlockSpec((N, BB, D), lambda i: (0, i, 0)),
        out_shape=jax.ShapeDtypeStruct((N, B, D), jnp.float32),
    )(char_comb, style_comb, W1, vec(b1), W2, vec(b2), vec(ln_g), vec(ln_b),
      xt)
    return jnp.transpose(out_t, (1, 0, 2))


def kernel(x, character_id, style_id, char_table, style_table, char_memory,
           style_memory, W1, b1, W2, b2, ln_g, ln_b):
    char_comb, style_comb = _sc_gather(
        character_id, style_id, char_table, style_table, char_memory,
        style_memory)
    return _tc_tail(char_comb, style_comb, W1, b1, W2, b2, ln_g, ln_b, x)
